# Initial kernel scaffold; baseline (speedup 1.0000x reference)
import jax
import jax.numpy as jnp
from jax.experimental import pallas as pl


def kernel(x, edge_index, W1, b1, Wm1, bm1, Wm2, bm2):
    raise NotImplementedError("write your pallas kernel here")


# speedup vs baseline: 88.3643x; 88.3643x over previous
import functools

import jax
import jax.numpy as jnp
from jax import lax
from jax.experimental import pallas as pl
from jax.experimental.pallas import tpu as pltpu
from jax.experimental.pallas import tpu_sc as plsc

N = 100000
E = 6400000
NOUT = N // 5
NW = 32
EPT = E // NW
CHUNK = 4000
N_CHUNKS = EPT // CHUNK
UNROLL = 10
INNER = CHUNK // (16 * UNROLL)

_mesh = plsc.VectorSubcoreMesh(core_axis_name="c", subcore_axis_name="s")


@functools.partial(
    pl.kernel,
    mesh=_mesh,
    out_type=jax.ShapeDtypeStruct((NW, NOUT), jnp.float32),
    compiler_params=pltpu.CompilerParams(needs_layout_passes=False),
    scratch_types=[
        pltpu.VMEM((N,), jnp.float32),
        pltpu.VMEM((NOUT,), jnp.float32),
        pltpu.VMEM((CHUNK,), jnp.int32),
        pltpu.VMEM((CHUNK,), jnp.int32),
    ],
)
def _edge_scatter(x_hbm, ei_hbm, part_hbm, x_v, acc_v, src_v, dst_v):
    wid = lax.axis_index("s") * 2 + lax.axis_index("c")

    pltpu.sync_copy(x_hbm, x_v)

    zeros16 = jnp.zeros((16,), jnp.float32)

    def zbody(i, carry):
        acc_v[pl.ds(i * 16, 16)] = zeros16
        return carry

    lax.fori_loop(0, NOUT // 16, zbody, 0, unroll=8)

    base0 = wid * EPT

    def chunk_body(c, carry):
        base = base0 + c * CHUNK
        pltpu.sync_copy(ei_hbm.at[pl.ds(base, CHUNK)], src_v)
        pltpu.sync_copy(ei_hbm.at[pl.ds(E + base, CHUNK)], dst_v)

        def inner(i, icarry):
            off = i * (16 * UNROLL)
            for u in range(UNROLL):
                o = off + u * 16
                s = src_v[pl.ds(o, 16)]
                d = dst_v[pl.ds(o, 16)]
                q = d // 5
                m = (q * 5) == d
                vals = plsc.load_gather(x_v, [s], mask=m)
                plsc.addupdate_scatter(acc_v, [q], vals, mask=m)
            return icarry

        lax.fori_loop(0, INNER, inner, 0)
        return carry

    lax.fori_loop(0, N_CHUNKS, chunk_body, 0)

    pltpu.sync_copy(acc_v, part_hbm.at[wid])


def _tail_body(p_ref, w1_ref, b1_ref, wm1_ref, bm1_ref, wm2_ref, bm2_ref, o_ref):
    s = jnp.sum(p_ref[...], axis=0)
    val = s * w1_ref[0, 0] + b1_ref[0]
    out = jnp.full((NOUT,), bm2_ref[0], jnp.float32)
    for j in range(4):
        cj = wm1_ref[0, j] + wm1_ref[1, j]
        out = out + jnp.maximum(val * cj + bm1_ref[j], 0.0) * wm2_ref[j, 0]
    o_ref[...] = out


_tail = pl.pallas_call(
    _tail_body,
    out_shape=jax.ShapeDtypeStruct((NOUT,), jnp.float32),
    in_specs=[
        pl.BlockSpec(memory_space=pltpu.VMEM),
        pl.BlockSpec(memory_space=pltpu.SMEM),
        pl.BlockSpec(memory_space=pltpu.SMEM),
        pl.BlockSpec(memory_space=pltpu.SMEM),
        pl.BlockSpec(memory_space=pltpu.SMEM),
        pl.BlockSpec(memory_space=pltpu.SMEM),
        pl.BlockSpec(memory_space=pltpu.SMEM),
    ],
    out_specs=pl.BlockSpec(memory_space=pltpu.VMEM),
)


def kernel(x, edge_index, W1, b1, Wm1, bm1, Wm2, bm2):
    xf = x.reshape(-1)
    partials = _edge_scatter(xf, edge_index.reshape(-1))
    return _tail(partials, W1, b1, Wm1, bm1, Wm2, bm2)


# speedup vs baseline: 321.0915x; 3.6337x over previous
import functools

import jax
import jax.numpy as jnp
from jax import lax
from jax.experimental import pallas as pl
from jax.experimental.pallas import tpu as pltpu
from jax.experimental.pallas import tpu_sc as plsc

N = 100000
E = 6400000
NOUT = N // 5
NW = 32
EPT = E // NW
CHUNK = 2000
N_CHUNKS = EPT // CHUNK
UNROLL = 5
INNER = CHUNK // (16 * UNROLL)
NBUF = 2

_mesh = plsc.VectorSubcoreMesh(core_axis_name="c", subcore_axis_name="s")


@functools.partial(
    pl.kernel,
    mesh=_mesh,
    out_type=jax.ShapeDtypeStruct((NW, NOUT), jnp.float32),
    compiler_params=pltpu.CompilerParams(needs_layout_passes=False),
    scratch_types=[
        pltpu.VMEM((N,), jnp.float32),
        pltpu.VMEM((NOUT,), jnp.float32),
        [pltpu.VMEM((CHUNK,), jnp.int32)] * NBUF,
        [pltpu.VMEM((CHUNK,), jnp.int32)] * NBUF,
        pltpu.SemaphoreType.DMA,
        [pltpu.SemaphoreType.DMA] * NBUF,
        [pltpu.SemaphoreType.DMA] * NBUF,
    ],
)
def _edge_scatter(x_hbm, ei_hbm, part_hbm, x_v, acc_v, src_v, dst_v,
                  x_sem, src_sems, dst_sems):
    wid = lax.axis_index("s") * 2 + lax.axis_index("c")
    base0 = wid * EPT

    def start_fetch(c, b):
        base = base0 + c * CHUNK
        pltpu.async_copy(ei_hbm.at[pl.ds(base, CHUNK)], src_v[b], src_sems[b])
        pltpu.async_copy(ei_hbm.at[pl.ds(E + base, CHUNK)], dst_v[b], dst_sems[b])

    def wait_fetch(b):
        pltpu.make_async_copy(ei_hbm.at[pl.ds(0, CHUNK)], src_v[b], src_sems[b]).wait()
        pltpu.make_async_copy(ei_hbm.at[pl.ds(0, CHUNK)], dst_v[b], dst_sems[b]).wait()

    x_copy = pltpu.async_copy(x_hbm, x_v, x_sem)
    for b in range(NBUF):
        start_fetch(b, b)

    zeros16 = jnp.zeros((16,), jnp.float32)

    def zbody(i, carry):
        acc_v[pl.ds(i * 16, 16)] = zeros16
        return carry

    lax.fori_loop(0, NOUT // 16, zbody, 0, unroll=8)
    x_copy.wait()

    def outer(g, carry):
        for b in range(NBUF):
            c = g * NBUF + b
            wait_fetch(b)

            def inner(i, icarry):
                off = i * (16 * UNROLL)
                for u in range(UNROLL):
                    o = off + u * 16
                    s = src_v[b][pl.ds(o, 16)]
                    d = dst_v[b][pl.ds(o, 16)]
                    t = d.astype(jnp.uint32) * jnp.uint32(3435973837)
                    m = t <= jnp.uint32(858993459)
                    q = plsc.bitcast(t, jnp.int32)
                    vals = plsc.load_gather(x_v, [s], mask=m)
                    plsc.addupdate_scatter(acc_v, [q], vals, mask=m)
                return icarry

            lax.fori_loop(0, INNER, inner, 0)

            @pl.when(c + NBUF < N_CHUNKS)
            def _():
                start_fetch(c + NBUF, b)

        return carry

    lax.fori_loop(0, N_CHUNKS // NBUF, outer, 0)

    pltpu.sync_copy(acc_v, part_hbm.at[wid])


def _bf(v):
    return v.astype(jnp.bfloat16).astype(jnp.float32)


def _tail_body(p_ref, w1_ref, b1_ref, wm1_ref, bm1_ref, wm2_ref, bm2_ref, o_ref):
    s = jnp.sum(p_ref[...], axis=0)
    val = s * w1_ref[0, 0] + b1_ref[0]
    vb = _bf(val)
    out = jnp.full((NOUT,), bm2_ref[0], jnp.float32)
    for j in range(4):
        cj = _bf(wm1_ref[0, j]) + _bf(wm1_ref[1, j])
        hj = jnp.maximum(vb * cj + bm1_ref[j], 0.0)
        out = out + _bf(hj) * _bf(wm2_ref[j, 0])
    o_ref[...] = out


_tail = pl.pallas_call(
    _tail_body,
    out_shape=jax.ShapeDtypeStruct((NOUT,), jnp.float32),
    in_specs=[
        pl.BlockSpec(memory_space=pltpu.VMEM),
        pl.BlockSpec(memory_space=pltpu.SMEM),
        pl.BlockSpec(memory_space=pltpu.SMEM),
        pl.BlockSpec(memory_space=pltpu.SMEM),
        pl.BlockSpec(memory_space=pltpu.SMEM),
        pl.BlockSpec(memory_space=pltpu.SMEM),
        pl.BlockSpec(memory_space=pltpu.SMEM),
    ],
    out_specs=pl.BlockSpec(memory_space=pltpu.VMEM),
)


def kernel(x, edge_index, W1, b1, Wm1, bm1, Wm2, bm2):
    xf = x.reshape(-1)
    partials = _edge_scatter(xf, edge_index.reshape(-1))
    return _tail(partials, W1, b1, Wm1, bm1, Wm2, bm2)


# speedup vs baseline: 790.1032x; 2.4607x over previous
import functools

import jax
import jax.numpy as jnp
from jax import lax
from jax.experimental import pallas as pl
from jax.experimental.pallas import tpu as pltpu
from jax.experimental.pallas import tpu_sc as plsc

N = 100000
E = 6400000
NOUT = N // 5
NW = 32
CHUNK = 2048
N_CHUNKS = E // CHUNK
MAXJ = -(-N_CHUNKS // NW)
UNROLL = 8
INNER = CHUNK // (16 * UNROLL)
NBUF = 2

_mesh = plsc.VectorSubcoreMesh(core_axis_name="c", subcore_axis_name="s")


@functools.partial(
    pl.kernel,
    mesh=_mesh,
    out_type=jax.ShapeDtypeStruct((NW, NOUT), jnp.float32),
    compiler_params=pltpu.CompilerParams(needs_layout_passes=False),
    scratch_types=[
        pltpu.VMEM((N,), jnp.float32),
        pltpu.VMEM((NOUT,), jnp.float32),
        [pltpu.VMEM((2, CHUNK), jnp.int32)] * NBUF,
        pltpu.SemaphoreType.DMA,
        [pltpu.SemaphoreType.DMA] * NBUF,
    ],
)
def _edge_scatter(x_hbm, ei_hbm, part_hbm, x_v, acc_v, ed_v, x_sem, ed_sems):
    wid = lax.axis_index("s") * 2 + lax.axis_index("c")

    def start_fetch(c, b):
        pltpu.async_copy(
            ei_hbm.at[pl.ds(0, 2), pl.ds(c * CHUNK, CHUNK)], ed_v[b], ed_sems[b]
        )

    def wait_fetch(b):
        pltpu.make_async_copy(
            ei_hbm.at[pl.ds(0, 2), pl.ds(0, CHUNK)], ed_v[b], ed_sems[b]
        ).wait()

    x_copy = pltpu.async_copy(x_hbm, x_v, x_sem)
    for b in range(NBUF):
        start_fetch(wid + NW * b, b)

    zeros16 = jnp.zeros((16,), jnp.float32)

    def zbody(i, carry):
        acc_v[pl.ds(i * 16, 16)] = zeros16
        return carry

    lax.fori_loop(0, NOUT // 16, zbody, 0, unroll=8)
    x_copy.wait()

    def outer(g, carry):
        for b in range(NBUF):
            j = g * NBUF + b
            c = wid + NW * j

            @pl.when(c < N_CHUNKS)
            def _():
                wait_fetch(b)

                def inner(i, icarry):
                    off = i * (16 * UNROLL)
                    ss, qs, ms = [], [], []
                    for u in range(UNROLL):
                        o = off + u * 16
                        d = ed_v[b][1, pl.ds(o, 16)]
                        s = ed_v[b][0, pl.ds(o, 16)]
                        t = d.astype(jnp.uint32) * jnp.uint32(3435973837)
                        ss.append(s)
                        ms.append(t <= jnp.uint32(858993459))
                        qs.append(plsc.bitcast(t, jnp.int32))
                    vals = [
                        plsc.load_gather(x_v, [ss[u]], mask=ms[u])
                        for u in range(UNROLL)
                    ]
                    for u in range(UNROLL):
                        plsc.addupdate_scatter(acc_v, [qs[u]], vals[u], mask=ms[u])
                    return icarry

                lax.fori_loop(0, INNER, inner, 0)

                nxt = c + NW * NBUF

                @pl.when(nxt < N_CHUNKS)
                def _():
                    start_fetch(nxt, b)

        return carry

    lax.fori_loop(0, MAXJ // NBUF, outer, 0)

    pltpu.sync_copy(acc_v, part_hbm.at[wid])


def _bf(v):
    return v.astype(jnp.bfloat16).astype(jnp.float32)


def _tail_body(p_ref, w1_ref, b1_ref, wm1_ref, bm1_ref, wm2_ref, bm2_ref, o_ref):
    s = jnp.sum(p_ref[...], axis=0)
    val = s * w1_ref[0, 0] + b1_ref[0]
    vb = _bf(val)
    out = jnp.full((NOUT,), bm2_ref[0], jnp.float32)
    for j in range(4):
        cj = _bf(wm1_ref[0, j]) + _bf(wm1_ref[1, j])
        hj = jnp.maximum(vb * cj + bm1_ref[j], 0.0)
        out = out + _bf(hj) * _bf(wm2_ref[j, 0])
    o_ref[...] = out


_tail = pl.pallas_call(
    _tail_body,
    out_shape=jax.ShapeDtypeStruct((NOUT,), jnp.float32),
    in_specs=[
        pl.BlockSpec(memory_space=pltpu.VMEM),
        pl.BlockSpec(memory_space=pltpu.SMEM),
        pl.BlockSpec(memory_space=pltpu.SMEM),
        pl.BlockSpec(memory_space=pltpu.SMEM),
        pl.BlockSpec(memory_space=pltpu.SMEM),
        pl.BlockSpec(memory_space=pltpu.SMEM),
        pl.BlockSpec(memory_space=pltpu.SMEM),
    ],
    out_specs=pl.BlockSpec(memory_space=pltpu.VMEM),
)


def kernel(x, edge_index, W1, b1, Wm1, bm1, Wm2, bm2):
    xf = x.reshape(-1)
    partials = _edge_scatter(xf, edge_index)
    return _tail(partials, W1, b1, Wm1, bm1, Wm2, bm2)


# speedup vs baseline: 959.3542x; 1.2142x over previous
import functools

import jax
import jax.numpy as jnp
from jax import lax
from jax.experimental import pallas as pl
from jax.experimental.pallas import tpu as pltpu
from jax.experimental.pallas import tpu_sc as plsc

N = 100000
E = 6400000
NOUT = N // 5
NW = 32
CHUNK = 1280
N_CHUNKS = E // CHUNK
MAXJ = -(-N_CHUNKS // NW)
UNROLL = 8
INNER = CHUNK // (16 * UNROLL)
NBUF = 4

_mesh = plsc.VectorSubcoreMesh(core_axis_name="c", subcore_axis_name="s")


@functools.partial(
    pl.kernel,
    mesh=_mesh,
    out_type=jax.ShapeDtypeStruct((NW, NOUT), jnp.float32),
    compiler_params=pltpu.CompilerParams(needs_layout_passes=False),
    scratch_types=[
        pltpu.VMEM((N,), jnp.float32),
        pltpu.VMEM((NOUT,), jnp.float32),
        [pltpu.VMEM((2, CHUNK), jnp.int32)] * NBUF,
        pltpu.SemaphoreType.DMA,
        [pltpu.SemaphoreType.DMA] * NBUF,
    ],
)
def _edge_scatter(x_hbm, ei_hbm, part_hbm, x_v, acc_v, ed_v, x_sem, ed_sems):
    wid = lax.axis_index("s") * 2 + lax.axis_index("c")

    def start_fetch(c, b):
        pltpu.async_copy(
            ei_hbm.at[pl.ds(0, 2), pl.ds(c * CHUNK, CHUNK)], ed_v[b], ed_sems[b]
        )

    def wait_fetch(b):
        pltpu.make_async_copy(
            ei_hbm.at[pl.ds(0, 2), pl.ds(0, CHUNK)], ed_v[b], ed_sems[b]
        ).wait()

    x_copy = pltpu.async_copy(x_hbm, x_v, x_sem)
    for b in range(NBUF):
        start_fetch(wid + NW * b, b)

    zeros16 = jnp.zeros((16,), jnp.float32)

    def zbody(i, carry):
        acc_v[pl.ds(i * 16, 16)] = zeros16
        return carry

    lax.fori_loop(0, NOUT // 16, zbody, 0, unroll=8)
    x_copy.wait()

    def outer(g, carry):
        for b in range(NBUF):
            j = g * NBUF + b
            c = wid + NW * j

            @pl.when(c < N_CHUNKS)
            def _():
                wait_fetch(b)

                def inner(i, icarry):
                    off = i * (16 * UNROLL)
                    ss, qs, ms = [], [], []
                    for u in range(UNROLL):
                        o = off + u * 16
                        d = ed_v[b][1, pl.ds(o, 16)]
                        s = ed_v[b][0, pl.ds(o, 16)]
                        t = d.astype(jnp.uint32) * jnp.uint32(3435973837)
                        ss.append(s)
                        ms.append(t <= jnp.uint32(858993459))
                        qs.append(plsc.bitcast(t, jnp.int32))
                    vals = [
                        plsc.load_gather(x_v, [ss[u]], mask=ms[u])
                        for u in range(UNROLL)
                    ]
                    for u in range(UNROLL):
                        plsc.addupdate_scatter(acc_v, [qs[u]], vals[u], mask=ms[u])
                    return icarry

                lax.fori_loop(0, INNER, inner, 0)

                nxt = c + NW * NBUF

                @pl.when(nxt < N_CHUNKS)
                def _():
                    start_fetch(nxt, b)

        return carry

    lax.fori_loop(0, -(-MAXJ // NBUF), outer, 0)

    pltpu.sync_copy(acc_v, part_hbm.at[wid])


def _bf(v):
    return v.astype(jnp.bfloat16).astype(jnp.float32)


def _tail_body(p_ref, w1_ref, b1_ref, wm1_ref, bm1_ref, wm2_ref, bm2_ref, o_ref):
    s = jnp.sum(p_ref[...], axis=0)
    val = s * w1_ref[0, 0] + b1_ref[0]
    vb = _bf(val)
    out = jnp.full((NOUT,), bm2_ref[0], jnp.float32)
    for j in range(4):
        cj = _bf(wm1_ref[0, j]) + _bf(wm1_ref[1, j])
        hj = jnp.maximum(vb * cj + bm1_ref[j], 0.0)
        out = out + _bf(hj) * _bf(wm2_ref[j, 0])
    o_ref[...] = out


_tail = pl.pallas_call(
    _tail_body,
    out_shape=jax.ShapeDtypeStruct((NOUT,), jnp.float32),
    in_specs=[
        pl.BlockSpec(memory_space=pltpu.VMEM),
        pl.BlockSpec(memory_space=pltpu.SMEM),
        pl.BlockSpec(memory_space=pltpu.SMEM),
        pl.BlockSpec(memory_space=pltpu.SMEM),
        pl.BlockSpec(memory_space=pltpu.SMEM),
        pl.BlockSpec(memory_space=pltpu.SMEM),
        pl.BlockSpec(memory_space=pltpu.SMEM),
    ],
    out_specs=pl.BlockSpec(memory_space=pltpu.VMEM),
)


def kernel(x, edge_index, W1, b1, Wm1, bm1, Wm2, bm2):
    xf = x.reshape(-1)
    partials = _edge_scatter(xf, edge_index)
    return _tail(partials, W1, b1, Wm1, bm1, Wm2, bm2)
